# Initial kernel scaffold; baseline (speedup 1.0000x reference)
import jax
import jax.numpy as jnp
from jax.experimental import pallas as pl


def kernel(history_items, history_mask, history_ratings, pos_item, title_table, feat_table, W1, b1, W2, b2, W3, b3, U1, ub1, U2, ub2):
    raise NotImplementedError("write your pallas kernel here")


# speedup vs baseline: 7.8837x; 7.8837x over previous
import functools

import jax
import jax.numpy as jnp
from jax import lax
from jax.experimental import pallas as pl
from jax.experimental.pallas import tpu as pltpu
from jax.experimental.pallas import tpu_sc as plsc

TEMP_INV = 1.0 / 0.07
B, L, V, TD, FD = 4096, 50, 100000, 384, 4
FDP = 16
HIST = B * L
NC, NS = 2, 16
NW = NC * NS
CH = 128
HIST_PER_W = HIST // NW
POS_PER_W = B // NW
N_HCHUNK = HIST_PER_W // CH


def _gather_sc(idx_hist, idx_pos, title_table, feat_flat):
    mesh = plsc.VectorSubcoreMesh(core_axis_name="c", subcore_axis_name="s")

    @functools.partial(
        pl.kernel,
        mesh=mesh,
        out_type=(
            jax.ShapeDtypeStruct((HIST, TD), jnp.float32),
            jax.ShapeDtypeStruct((FD, HIST), jnp.float32),
            jax.ShapeDtypeStruct((B, TD), jnp.float32),
            jax.ShapeDtypeStruct((FD, B), jnp.float32),
        ),
        scratch_types=[
            pltpu.VMEM((CH,), jnp.int32),
            pltpu.VMEM((FD, CH), jnp.int32),
            pltpu.VMEM((CH, TD), jnp.float32),
            pltpu.VMEM((FD, CH), jnp.float32),
            pltpu.SemaphoreType.DMA,
            pltpu.SemaphoreType.DMA,
        ],
    )
    def k(ih_hbm, ip_hbm, tt_hbm, ft_hbm, oht, ohf, opt, opf, idx_v, idxf_v,
          rows_v, featc_v, sem_t, sem_f):
        wid = lax.axis_index("s") * NC + lax.axis_index("c")

        def do_chunk(idx_src, base, out_t, out_f):
            pltpu.sync_copy(idx_src.at[pl.ds(base, CH)], idx_v)
            cp_t = pltpu.async_copy(tt_hbm.at[idx_v], rows_v, sem_t)
            for j in range(FD):
                for q in range(CH // 16):
                    s = idx_v[pl.ds(q * 16, 16)]
                    idxf_v[j, pl.ds(q * 16, 16)] = s * FD + j
            cps_f = [
                pltpu.async_copy(ft_hbm.at[idxf_v.at[j]], featc_v.at[j], sem_f)
                for j in range(FD)
            ]
            cp_t.wait()
            for cp in cps_f:
                cp.wait()
            pltpu.sync_copy(rows_v, out_t.at[pl.ds(base, CH)])
            for j in range(FD):
                pltpu.sync_copy(featc_v.at[j], out_f.at[j, pl.ds(base, CH)])

        hbase = wid * HIST_PER_W

        def body(g, carry):
            do_chunk(ih_hbm, hbase + g * CH, oht, ohf)
            return carry

        lax.fori_loop(0, N_HCHUNK, body, 0)
        do_chunk(ip_hbm, wid * POS_PER_W, opt, opf)

    return k(idx_hist, idx_pos, title_table, feat_flat)


def _item_tower_block(x, c, W1t, b1, W2, b2, W3, b3):
    h = x @ W1t + c + b1
    h = jnp.maximum(h, 0.0)
    h = h @ W2 + b2
    h = jnp.maximum(h, 0.0)
    e = h @ W3 + b3
    n = jnp.sqrt(jnp.sum(e * e, axis=-1, keepdims=True))
    return e / jnp.maximum(n, 1e-12)


def _tower_pool_body(g_ref, f_ref, r_ref, m_ref, W1t_ref, W1f_ref, b1_ref,
                     W2_ref, b2_ref, W3_ref, b3_ref, out_ref):
    bb = g_ref.shape[1]
    x = g_ref[...].reshape(L * bb, TD)
    W1f = W1f_ref[...]
    c3 = f_ref[0][:, :, None] * W1f[0][None, None, :]
    for j in range(1, FD):
        c3 = c3 + f_ref[j][:, :, None] * W1f[j][None, None, :]
    e = _item_tower_block(x, c3.reshape(L * bb, 256), W1t_ref[...],
                          b1_ref[...], W2_ref[...], b2_ref[...], W3_ref[...],
                          b3_ref[...])
    e3 = e.reshape(L, bb, 64)
    w = r_ref[...] * m_ref[...]
    wn = w / (jnp.sum(w, axis=0, keepdims=True) + 1e-8)
    out_ref[...] = jnp.sum(e3 * wn[:, :, None], axis=0)


def _pos_tower_body(g_ref, f_ref, W1t_ref, W1f_ref, b1_ref, W2_ref, b2_ref,
                    W3_ref, b3_ref, out_ref):
    W1f = W1f_ref[...]
    c = f_ref[0][:, None] * W1f[0][None, :]
    for j in range(1, FD):
        c = c + f_ref[j][:, None] * W1f[j][None, :]
    out_ref[...] = _item_tower_block(
        g_ref[...], c, W1t_ref[...], b1_ref[...],
        W2_ref[...], b2_ref[...], W3_ref[...], b3_ref[...])


def _final_body(p_ref, pe_ref, U1_ref, ub1_ref, U2_ref, ub2_ref, out_ref):
    h = jnp.maximum(p_ref[...] @ U1_ref[...] + ub1_ref[...], 0.0)
    u = h @ U2_ref[...] + ub2_ref[...]
    n = jnp.sqrt(jnp.sum(u * u, axis=-1, keepdims=True))
    u = u / jnp.maximum(n, 1e-12)
    out_ref[...] = (u @ pe_ref[...]) * TEMP_INV


def _full(spec):
    return pl.BlockSpec(spec, lambda i: tuple(0 for _ in spec))


def _tower_pool(g3, f3, rT, mT, W1t, W1f, b1, W2, b2, W3, b3):
    BB = 128
    grid = B // BB
    return pl.pallas_call(
        _tower_pool_body,
        grid=(grid,),
        in_specs=[
            pl.BlockSpec((L, BB, TD), lambda i: (0, i, 0)),
            pl.BlockSpec((FD, L, BB), lambda i: (0, 0, i)),
            pl.BlockSpec((L, BB), lambda i: (0, i)),
            pl.BlockSpec((L, BB), lambda i: (0, i)),
            _full((TD, 256)), _full((FD, 256)), _full((256,)),
            _full((256, 128)), _full((128,)),
            _full((128, 64)), _full((64,)),
        ],
        out_specs=pl.BlockSpec((BB, 64), lambda i: (i, 0)),
        out_shape=jax.ShapeDtypeStruct((B, 64), jnp.float32),
    )(g3, f3, rT, mT, W1t, W1f, b1, W2, b2, W3, b3)


def _pos_tower(gp, fp, W1t, W1f, b1, W2, b2, W3, b3):
    BB = 512
    return pl.pallas_call(
        _pos_tower_body,
        grid=(B // BB,),
        in_specs=[
            pl.BlockSpec((BB, TD), lambda i: (i, 0)),
            pl.BlockSpec((FD, BB), lambda i: (0, i)),
            _full((TD, 256)), _full((FD, 256)), _full((256,)),
            _full((256, 128)), _full((128,)),
            _full((128, 64)), _full((64,)),
        ],
        out_specs=pl.BlockSpec((BB, 64), lambda i: (i, 0)),
        out_shape=jax.ShapeDtypeStruct((B, 64), jnp.float32),
    )(gp, fp, W1t, W1f, b1, W2, b2, W3, b3)


def _final(pooled, pos_emb_t, U1, ub1, U2, ub2):
    BB = 512
    return pl.pallas_call(
        _final_body,
        grid=(B // BB,),
        in_specs=[
            pl.BlockSpec((BB, 64), lambda i: (i, 0)),
            _full((64, B)),
            _full((64, 128)), _full((128,)),
            _full((128, 64)), _full((64,)),
        ],
        out_specs=pl.BlockSpec((BB, B), lambda i: (i, 0)),
        out_shape=jax.ShapeDtypeStruct((B, B), jnp.float32),
    )(pooled, pos_emb_t, U1, ub1, U2, ub2)


def kernel(history_items, history_mask, history_ratings, pos_item, title_table,
           feat_table, W1, b1, W2, b2, W3, b3, U1, ub1, U2, ub2):
    idx_hist = history_items.astype(jnp.int32).T.reshape(-1)
    idx_pos = pos_item.astype(jnp.int32)
    W1t = W1[:TD]
    W1f = W1[TD:]

    ht, hf, pt, pf = _gather_sc(idx_hist, idx_pos, title_table,
                                feat_table.reshape(-1))

    g3 = ht.reshape(L, B, TD)
    f3 = hf.reshape(FD, L, B)
    rT = history_ratings.T
    mT = history_mask.T

    pooled = _tower_pool(g3, f3, rT, mT, W1t, W1f, b1, W2, b2, W3, b3)
    pos_emb = _pos_tower(pt, pf, W1t, W1f, b1, W2, b2, W3, b3)
    return _final(pooled, pos_emb.T, U1, ub1, U2, ub2)
